# baseline (speedup 1.0000x reference)
import functools

import jax
import jax.numpy as jnp
from jax import lax
from jax.experimental import pallas as pl
from jax.experimental.pallas import tpu as pltpu
from jax.experimental.pallas import tpu_sc as plsc

_VOCAB = 1000000
_EMBED = 64
_B = 4096
_N = 200

_NC = 2
_NS = 16
_NW = _NC * _NS
_BPW = _B // _NW
_CH = _BPW
_JC = 8
_TC = _N // _JC

_mesh = plsc.VectorSubcoreMesh(core_axis_name="c", subcore_axis_name="s")


@functools.partial(
    pl.kernel,
    mesh=_mesh,
    compiler_params=pltpu.CompilerParams(use_tc_tiling_on_sc=False, needs_layout_passes=False),
    out_type=jax.ShapeDtypeStruct((_B, _N, _EMBED), jnp.float32),
    scratch_types=[
        pltpu.VMEM((_BPW * _N,), jnp.int32),
        pltpu.VMEM((_N, _BPW), jnp.int32),
        pltpu.VMEM((2, _CH, _EMBED), jnp.float32),
        pltpu.VMEM((2, _CH, _EMBED), jnp.float32),
        pltpu.VMEM((_N, _EMBED), jnp.float32),
        pltpu.SemaphoreType.DMA,
        pltpu.SemaphoreType.DMA,
        pltpu.SemaphoreType.DMA,
        pltpu.SemaphoreType.DMA,
    ],
)
def _embed_sc(x_hbm, tok_hbm, pos_hbm, out_hbm, slab_v, idxT_v, grows_v, orows_v,
              pos_v, gsem0, gsem1, osem0, osem1):
    cid = lax.axis_index("c")
    sid = lax.axis_index("s")
    wid = sid * _NC + cid
    b0 = wid * _BPW

    pltpu.sync_copy(pos_hbm, pos_v)
    pltpu.sync_copy(x_hbm.at[pl.ds(b0 * _N, _BPW * _N)], slab_v)

    prem = lax.iota(jnp.int32, 16) * _N

    def tr_body(n, carry):
        for k in range(_BPW // 16):
            ivec = prem + (k * 16 * _N + n)
            idxT_v[n, pl.ds(16 * k, 16)] = plsc.load_gather(slab_v, [ivec])
        return carry

    lax.fori_loop(0, _N, tr_body, 0)

    def gsem(b):
        return gsem0 if b == 0 else gsem1

    def osem(b):
        return osem0 if b == 0 else osem1

    def fire_gather(t, j):
        b = j % 2
        pltpu.async_copy(tok_hbm.at[idxT_v.at[t * _JC + j]], grows_v.at[b], gsem(b))

    def out_slice(t, j):
        return out_hbm.at[pl.ds(b0, _BPW), t * _JC + j]

    def slot(t, j, do_outwait, do_fire):
        b = j % 2
        n = t * _JC + j
        pltpu.make_async_copy(
            tok_hbm.at[idxT_v.at[n]], grows_v.at[b], gsem(b)
        ).wait()
        if do_outwait:
            j3 = (j - 2) % _JC
            t3 = t - 1 if j < 2 else t
            pltpu.make_async_copy(orows_v.at[b], out_slice(t3, j3), osem(b)).wait()
        prow = [pos_v[n, pl.ds(16 * d, 16)] for d in range(4)]

        def add_body(i, c):
            for ii in range(8):
                row = i * 8 + ii
                for d in range(4):
                    sl = pl.ds(16 * d, 16)
                    orows_v[b, row, sl] = grows_v[b, row, sl] + prow[d]
            return c

        lax.fori_loop(0, _CH // 8, add_body, 0)
        pltpu.async_copy(orows_v.at[b], out_slice(t, j), osem(b))
        if do_fire:
            j2 = (j + 2) % _JC
            t2 = t + 1 if j >= _JC - 2 else t
            fire_gather(t2, j2)

    fire_gather(0, 0)
    fire_gather(0, 1)

    for j in range(_JC):
        slot(0, j, do_outwait=(j >= 2), do_fire=True)

    def t_body(t, carry):
        for j in range(_JC):
            slot(t, j, do_outwait=True, do_fire=True)
        return carry

    lax.fori_loop(1, _TC - 1, t_body, 0)

    for j in range(_JC):
        slot(_TC - 1, j, do_outwait=True, do_fire=(j < _JC - 2))

    pltpu.make_async_copy(
        orows_v.at[0], out_slice(_TC - 1, _JC - 2), osem0
    ).wait()
    pltpu.make_async_copy(
        orows_v.at[1], out_slice(_TC - 1, _JC - 1), osem1
    ).wait()


def kernel(x, tok_table, pos_table):
    return _embed_sc(x.astype(jnp.int32).reshape(-1), tok_table, pos_table)
